# baseline (speedup 1.0000x reference)
import functools

import jax
import jax.numpy as jnp
from jax.experimental import pallas as pl
from jax.experimental.pallas import tpu as pltpu


def _mlp_body(col_ref, x_ref, w1_ref, b1p_ref, w2p_ref, b2_ref, o_ref,
              h_ref, *, nff):
    j = pl.program_id(1)
    slot = jax.lax.rem(j, 2)
    prev = jax.lax.rem(j + 1, 2)

    x = x_ref[...].astype(jnp.bfloat16)
    h_ref[slot] = jnp.dot(x, w1_ref[0].astype(jnp.bfloat16),
                          preferred_element_type=jnp.float32)

    g = jax.nn.gelu(h_ref[prev] + b1p_ref[0, 0]).astype(jnp.bfloat16)
    acc = jnp.dot(g, w2p_ref[0].astype(jnp.bfloat16),
                  preferred_element_type=jnp.float32)

    @pl.when(j == 1)
    def _init():
        o_ref[...] = acc + b2_ref[0, 0]

    @pl.when(j > 1)
    def _accum():
        o_ref[...] += acc


@functools.partial(jax.jit, static_argnames=("bt", "bf"))
def _moe_mlp(hidden_states, W1, b1, W2, b2, col, bt=1024, bf=1024):
    T, D = hidden_states.shape
    E, _, F = W1.shape
    nff = F // bf
    assert nff >= 2
    col_arr = jnp.atleast_1d(jnp.asarray(col, jnp.int32))
    b1r = b1.reshape(E, nff, 1, bf)
    b2r = b2.reshape(E, 1, 1, D)

    grid = (T // bt, nff + 1)
    grid_spec = pltpu.PrefetchScalarGridSpec(
        num_scalar_prefetch=1,
        grid=grid,
        in_specs=[
            pl.BlockSpec((bt, D), lambda i, j, c: (i, 0)),
            pl.BlockSpec((1, D, bf),
                         lambda i, j, c: (c[0], 0, jnp.minimum(j, nff - 1))),
            pl.BlockSpec((1, 1, 1, bf),
                         lambda i, j, c: (c[0], jnp.maximum(j - 1, 0), 0, 0)),
            pl.BlockSpec((1, bf, D),
                         lambda i, j, c: (c[0], jnp.maximum(j - 1, 0), 0)),
            pl.BlockSpec((1, 1, 1, D), lambda i, j, c: (c[0], 0, 0, 0)),
        ],
        out_specs=pl.BlockSpec((bt, D), lambda i, j, c: (i, 0)),
        scratch_shapes=[pltpu.VMEM((2, bt, bf), jnp.float32)],
    )
    body = functools.partial(_mlp_body, nff=nff)
    return pl.pallas_call(
        body,
        grid_spec=grid_spec,
        out_shape=jax.ShapeDtypeStruct((T, D), jnp.float32),
        compiler_params=pltpu.CompilerParams(
            dimension_semantics=("parallel", "arbitrary"),
        ),
    )(col_arr, hidden_states, W1, b1r, W2, b2r)


def kernel(hidden_states, W1, b1, W2, b2, col):
    return _moe_mlp(hidden_states, W1, b1, W2, b2, col)


# speedup vs baseline: 1.2270x; 1.2270x over previous
import functools

import jax
import jax.numpy as jnp
from jax.experimental import pallas as pl
from jax.experimental.pallas import tpu as pltpu


def _mlp_body(col_ref, x_ref, w1_ref, b1_ref, w2_ref, b2_ref, o_ref):
    x = x_ref[...].astype(jnp.bfloat16)
    h = jnp.dot(x, w1_ref[0].astype(jnp.bfloat16),
                preferred_element_type=jnp.float32)
    h = jax.nn.gelu(h + b1_ref[0, 0]).astype(jnp.bfloat16)
    acc = jnp.dot(h, w2_ref[0].astype(jnp.bfloat16),
                  preferred_element_type=jnp.float32)
    j = pl.program_id(1)

    @pl.when(j == 0)
    def _init():
        o_ref[...] = acc + b2_ref[0, 0]

    @pl.when(j != 0)
    def _accum():
        o_ref[...] += acc


@functools.partial(jax.jit, static_argnames=("bt", "bf"))
def _moe_mlp(hidden_states, W1, b1, W2, b2, col, bt=1024, bf=1024):
    T, D = hidden_states.shape
    E, _, F = W1.shape
    col_arr = jnp.atleast_1d(jnp.asarray(col, jnp.int32))
    b1r = b1.reshape(E, F // bf, 1, bf)
    b2r = b2.reshape(E, 1, 1, D)

    grid = (T // bt, F // bf)
    grid_spec = pltpu.PrefetchScalarGridSpec(
        num_scalar_prefetch=1,
        grid=grid,
        in_specs=[
            pl.BlockSpec((bt, D), lambda i, j, c: (i, 0)),
            pl.BlockSpec((1, D, bf), lambda i, j, c: (c[0], 0, j)),
            pl.BlockSpec((1, 1, 1, bf), lambda i, j, c: (c[0], j, 0, 0)),
            pl.BlockSpec((1, bf, D), lambda i, j, c: (c[0], j, 0)),
            pl.BlockSpec((1, 1, 1, D), lambda i, j, c: (c[0], 0, 0, 0)),
        ],
        out_specs=pl.BlockSpec((bt, D), lambda i, j, c: (i, 0)),
    )
    return pl.pallas_call(
        _mlp_body,
        grid_spec=grid_spec,
        out_shape=jax.ShapeDtypeStruct((T, D), jnp.float32),
        compiler_params=pltpu.CompilerParams(
            dimension_semantics=("parallel", "arbitrary"),
        ),
    )(col_arr, hidden_states, W1, b1r, W2, b2r)


def kernel(hidden_states, W1, b1, W2, b2, col):
    return _moe_mlp(hidden_states, W1, b1, W2, b2, col)
